# baseline (speedup 1.0000x reference)
import jax
import jax.numpy as jnp
from jax import lax
from jax.experimental import pallas as pl
from jax.experimental.pallas import tpu as pltpu
from jax.experimental.pallas import tpu_sc as plsc

_B = 16384
_CHUNK = 128


def _sc_gather(idx_all, tabs):
    info = plsc.get_sparse_core_info()
    nw = info.num_cores * info.num_subcores
    b_per_w = _B // nw
    n_chunks = b_per_w // _CHUNK
    n_tabs = len(tabs)

    mesh = plsc.VectorSubcoreMesh(core_axis_name="c", subcore_axis_name="s")

    def body(*refs):
        idx_hbm = refs[0]
        tab_hbms = refs[1:1 + n_tabs]
        outs = refs[1 + n_tabs:1 + 2 * n_tabs]
        idx_v, rows_v, sem = refs[1 + 2 * n_tabs:]
        wid = lax.axis_index("s") * info.num_cores + lax.axis_index("c")
        base = wid * b_per_w
        pltpu.sync_copy(idx_hbm.at[wid], idx_v)
        out = pl.ds(base, b_per_w)
        for t in range(n_tabs):
            copies = []
            for j in range(n_chunks):
                copies.append(pltpu.async_copy(
                    tab_hbms[t].at[idx_v.at[n_chunks * t + j]],
                    rows_v.at[pl.ds(j * _CHUNK, _CHUNK)], sem))
            for c in copies:
                c.wait()
            pltpu.sync_copy(rows_v, outs[t].at[out])

    f32 = jnp.float32
    run = pl.kernel(
        body,
        out_type=tuple(jax.ShapeDtypeStruct((_B, 128), f32)
                       for _ in range(n_tabs)),
        mesh=mesh,
        scratch_types=[
            pltpu.VMEM((n_chunks * n_tabs, _CHUNK), jnp.int32),
            pltpu.VMEM((b_per_w, 128), f32),
            pltpu.SemaphoreType.DMA,
        ],
    )
    return run(idx_all, *tabs)


_S_BIG = 253952
_S_ZIP = 6656
_W_BIG = 4096
_W_ZIP = 512


def _stack_transpose(p_refs, eye_ref, s, w, n):
    k = len(p_refs)
    pieces = [r[...] for r in p_refs]
    bound = n - (k - 1) * s - w * pl.program_id(0)
    cols = jax.lax.broadcasted_iota(jnp.int32, pieces[-1].shape, 1)
    pieces[-1] = jnp.where(cols < bound, pieces[-1], 0.0)
    x = jnp.concatenate(pieces, axis=0)
    return jax.lax.dot_general(x, eye_ref[...], (((0,), (0,)), ((), ())),
                               preferred_element_type=jnp.float32)


def _tp_big_body(*args):
    pus, eye_ref, ou_ref = args[0:4], args[4], args[5]
    ou_ref[...] = _stack_transpose(pus, eye_ref, _S_BIG, _W_BIG, 1000000)


def _tp_zip_body(*args):
    pzs, eye_ref, oz_ref = args[0:16], args[16], args[17]
    oz_ref[...] = _stack_transpose(pzs, eye_ref, _S_ZIP, _W_ZIP, 100000)


def _transpose_big(table, interpret=False):
    f32 = jnp.float32
    pt = table.T
    eye = jnp.eye(128, dtype=f32)
    s, w = _S_BIG, _W_BIG
    grid = s // w
    last_blk = (1000000 - 1) // w

    def pick_block(a):
        return pl.BlockSpec(
            (32, w), lambda i, a=a: (0, jnp.minimum((s // w) * a + i, last_blk)))

    full = lambda shape: pl.BlockSpec(shape, lambda i: (0, 0))
    out, = pl.pallas_call(
        _tp_big_body,
        grid=(grid,),
        in_specs=[pick_block(a) for a in range(4)] + [full((128, 128))],
        out_specs=[pl.BlockSpec((w, 128), lambda i: (i, 0))],
        out_shape=[jax.ShapeDtypeStruct((s, 128), f32)],
        interpret=interpret,
    )(pt, pt, pt, pt, eye)
    return out


def _transpose_zip(zip_emb, interpret=False):
    f32 = jnp.float32
    pz = zip_emb.T
    eye = jnp.eye(128, dtype=f32)
    sz, wz = _S_ZIP, _W_ZIP
    gz = sz // wz
    last_z = (100000 - 1) // wz

    def pick_z(a):
        return pl.BlockSpec(
            (8, wz), lambda i, a=a: (0, jnp.minimum((sz // wz) * a + i, last_z)))

    full = lambda shape: pl.BlockSpec(shape, lambda i: (0, 0))
    ztab2, = pl.pallas_call(
        _tp_zip_body,
        grid=(gz,),
        in_specs=[pick_z(a) for a in range(16)] + [full((128, 128))],
        out_specs=[pl.BlockSpec((wz, 128), lambda i: (i, 0))],
        out_shape=[jax.ShapeDtypeStruct((sz, 128), f32)],
        interpret=interpret,
    )(*([pz] * 16), eye)
    return ztab2


def _tc_body(g_u, g_i, g_z, hist, sidx,
             age_t, gen_t, occ_t, uctx_t, genre_t, year_t, ictx_t,
             w1u_big, w1z_big, w1u_age, w1u_gen, w1u_occ, w1u_ctx, w1u_hist,
             hist_W, hist_b, u_b1, u_W2, u_b2,
             w1i_big, w1i_genre, w1i_year, w1i_ictx, i_b1, i_W2, i_b2,
             out):
    f32 = jnp.float32
    bs = g_u.shape[0]
    idx = sidx[...]
    lane = lax.broadcasted_iota(jnp.int32, (bs, 128), 1)
    grp32 = lane >> 5
    grp8 = lane >> 3

    def emb(k, tab_ref, w_ref):
        n = tab_ref.shape[0]
        col = idx[:, k:k + 1]
        oh = (col == lax.broadcasted_iota(jnp.int32, (bs, n), 1)).astype(f32)
        fused = jnp.dot(tab_ref[...], w_ref[...], preferred_element_type=f32)
        return jnp.dot(oh, fused, preferred_element_type=f32)

    def pick(g_ref, grp, k):
        m = (grp == idx[:, k:k + 1]).astype(f32)
        return g_ref[...] * m

    h = jnp.dot(hist[...], hist_W[...], preferred_element_type=f32) + hist_b[...]
    au = jnp.dot(pick(g_u, grp32, 7), w1u_big[...], preferred_element_type=f32)
    au += jnp.dot(pick(g_z, grp8, 9), w1z_big[...], preferred_element_type=f32)
    au += emb(0, age_t, w1u_age)
    au += emb(1, gen_t, w1u_gen)
    au += emb(2, occ_t, w1u_occ)
    au += emb(3, uctx_t, w1u_ctx)
    au += jnp.dot(h, w1u_hist[...], preferred_element_type=f32)
    au += u_b1[...]
    au = jnp.maximum(au, 0.0)
    u = jnp.maximum(jnp.dot(au, u_W2[...], preferred_element_type=f32) + u_b2[...], 0.0)

    ai = jnp.dot(pick(g_i, grp32, 8), w1i_big[...], preferred_element_type=f32)
    ai += emb(4, genre_t, w1i_genre)
    ai += emb(5, year_t, w1i_year)
    ai += emb(6, ictx_t, w1i_ictx)
    ai += i_b1[...]
    ai = jnp.maximum(ai, 0.0)
    it = jnp.maximum(jnp.dot(ai, i_W2[...], preferred_element_type=f32) + i_b2[...], 0.0)

    ones = jnp.full((u.shape[1], 1), 1.0, f32)
    out[...] = jnp.dot(u * it, ones, preferred_element_type=f32)


def _tc_towers(g_u, g_i, g_z, history_emb, sidx, small_tabs, weights,
               bs=2048, interpret=False):
    grid = _B // bs
    f32 = jnp.float32

    def batched(d):
        return pl.BlockSpec((bs, d), lambda i: (i, 0))

    def full2(shape):
        return pl.BlockSpec(shape, lambda i: (0, 0))

    in_specs = (
        [batched(128), batched(128), batched(128), batched(history_emb.shape[1]),
         batched(16)]
        + [full2(t.shape) for t in small_tabs]
        + [full2(w.shape) for w in weights]
    )
    out_spec = pl.BlockSpec((bs, 1), lambda i: (i, 0))
    return pl.pallas_call(
        _tc_body,
        grid=(grid,),
        in_specs=in_specs,
        out_specs=out_spec,
        out_shape=jax.ShapeDtypeStruct((_B, 1), f32),
        interpret=interpret,
    )(g_u, g_i, g_z, history_emb, sidx, *small_tabs, *weights)


def kernel(user_id, age, gender, occupation, zipcode, context, item_id, genre,
           year, item_context, history_emb,
           user_id_emb, age_emb, gender_emb, occupation_emb, zip_emb,
           u_context_emb, hist_W, hist_b, u_W1, u_b1, u_W2, u_b2,
           item_id_emb, genre_emb, year_emb, i_context_emb,
           i_W1, i_b1, i_W2, i_b2):
    i32 = jnp.int32
    uid = user_id.astype(i32)
    iid = item_id.astype(i32)
    zid = zipcode.astype(i32)

    ztab2 = _transpose_zip(zip_emb)
    utab2 = _transpose_big(user_id_emb)

    chunks = lambda a: a.reshape(32, 4, _CHUNK)
    idx_uz = jnp.concatenate([chunks(uid % _S_BIG), chunks(zid % _S_ZIP)], axis=1)
    g_u, g_z = _sc_gather(idx_uz, [utab2, ztab2])
    itab2 = _transpose_big(item_id_emb)
    g_i, = _sc_gather(chunks(iid % _S_BIG), [itab2])

    sidx = jnp.stack(
        [age.astype(i32), gender.astype(i32), occupation.astype(i32),
         context.astype(i32), genre.astype(i32), year.astype(i32),
         item_context.astype(i32), uid // _S_BIG, iid // _S_BIG, zid // _S_ZIP,
         jnp.zeros((_B,), i32), jnp.zeros((_B,), i32),
         jnp.zeros((_B,), i32), jnp.zeros((_B,), i32),
         jnp.zeros((_B,), i32), jnp.zeros((_B,), i32)], axis=1)

    small_tabs = [age_emb, gender_emb, occupation_emb, u_context_emb,
                  genre_emb, year_emb, i_context_emb]
    weights = [
        jnp.tile(u_W1[0:32], (4, 1)),
        jnp.tile(u_W1[52:60], (16, 1)),
        u_W1[32:40], u_W1[40:44], u_W1[44:52], u_W1[60:68], u_W1[68:84],
        hist_W, hist_b.reshape(1, -1), u_b1.reshape(1, -1), u_W2,
        u_b2.reshape(1, -1),
        jnp.tile(i_W1[0:32], (4, 1)),
        i_W1[32:40], i_W1[40:44], i_W1[44:52],
        i_b1.reshape(1, -1), i_W2, i_b2.reshape(1, -1),
    ]
    out = _tc_towers(g_u, g_i, g_z, history_emb, sidx, small_tabs, weights)
    return out.reshape(_B)


# speedup vs baseline: 1.1221x; 1.1221x over previous
import jax
import jax.numpy as jnp
from jax import lax
from jax.experimental import pallas as pl
from jax.experimental.pallas import tpu as pltpu
from jax.experimental.pallas import tpu_sc as plsc

_B = 16384
_CHUNK = 128


def _sc_gather(idx_all, tabs):
    info = plsc.get_sparse_core_info()
    nw = info.num_cores * info.num_subcores
    b_per_w = _B // nw
    n_chunks = b_per_w // _CHUNK
    n_tabs = len(tabs)
    idx_rows = idx_all.shape[1]

    mesh = plsc.VectorSubcoreMesh(core_axis_name="c", subcore_axis_name="s")

    def body(*refs):
        idx_hbm = refs[0]
        tab_hbms = refs[1:1 + n_tabs]
        outs = refs[1 + n_tabs:1 + 2 * n_tabs]
        idx_v, rows_v, sem = refs[1 + 2 * n_tabs:]
        wid = lax.axis_index("s") * info.num_cores + lax.axis_index("c")
        base = wid * b_per_w
        pltpu.sync_copy(idx_hbm.at[wid], idx_v)
        out = pl.ds(base, b_per_w)
        for t in range(n_tabs):
            copies = []
            for j in range(n_chunks):
                copies.append(pltpu.async_copy(
                    tab_hbms[t].at[idx_v.at[n_chunks * t + j]],
                    rows_v.at[pl.ds(j * _CHUNK, _CHUNK)], sem))
            for c in copies:
                c.wait()
            pltpu.sync_copy(rows_v, outs[t].at[out])

    f32 = jnp.float32
    run = pl.kernel(
        body,
        out_type=tuple(jax.ShapeDtypeStruct((_B, 128), f32)
                       for _ in range(n_tabs)),
        mesh=mesh,
        scratch_types=[
            pltpu.VMEM((idx_rows, _CHUNK), jnp.int32),
            pltpu.VMEM((b_per_w, 128), f32),
            pltpu.SemaphoreType.DMA,
        ],
    )
    return run(idx_all, *tabs)


_S_BIG = 253952
_S_ZIP = 6656
_W_BIG = 4096
_W_ZIP = 512


def _stack_transpose(p_refs, eye_ref, s, w, n):
    k = len(p_refs)
    pieces = [r[...] for r in p_refs]
    bound = n - (k - 1) * s - w * pl.program_id(0)
    cols = jax.lax.broadcasted_iota(jnp.int32, pieces[-1].shape, 1)
    pieces[-1] = jnp.where(cols < bound, pieces[-1], 0.0)
    x = jnp.concatenate(pieces, axis=0)
    return jax.lax.dot_general(x, eye_ref[...], (((0,), (0,)), ((), ())),
                               preferred_element_type=jnp.float32)


def _tp_big_body(*args):
    pus, pis = args[0:4], args[4:8]
    eye_ref, ou_ref, oi_ref = args[8], args[9], args[10]
    ou_ref[...] = _stack_transpose(pus, eye_ref, _S_BIG, _W_BIG, 1000000)
    oi_ref[...] = _stack_transpose(pis, eye_ref, _S_BIG, _W_BIG, 1000000)


def _tp_zip_body(*args):
    pzs, eye_ref, oz_ref = args[0:16], args[16], args[17]
    oz_ref[...] = _stack_transpose(pzs, eye_ref, _S_ZIP, _W_ZIP, 100000)


def _transpose_big(utab, itab, interpret=False):
    f32 = jnp.float32
    pu = utab.T
    pi = itab.T
    eye = jnp.eye(128, dtype=f32)
    s, w = _S_BIG, _W_BIG
    grid = s // w
    last_blk = (1000000 - 1) // w

    def pick_block(a):
        return pl.BlockSpec(
            (32, w), lambda i, a=a: (0, jnp.minimum((s // w) * a + i, last_blk)))

    full = lambda shape: pl.BlockSpec(shape, lambda i: (0, 0))
    out = pl.pallas_call(
        _tp_big_body,
        grid=(grid,),
        in_specs=[pick_block(a) for a in range(4)] * 2 + [full((128, 128))],
        out_specs=[pl.BlockSpec((w, 128), lambda i: (i, 0)),
                   pl.BlockSpec((w, 128), lambda i: (i, 0))],
        out_shape=[jax.ShapeDtypeStruct((s, 128), f32),
                   jax.ShapeDtypeStruct((s, 128), f32)],
        interpret=interpret,
    )(pu, pu, pu, pu, pi, pi, pi, pi, eye)
    return out


def _transpose_zip(zip_emb, interpret=False):
    f32 = jnp.float32
    pz = zip_emb.T
    eye = jnp.eye(128, dtype=f32)
    sz, wz = _S_ZIP, _W_ZIP
    gz = sz // wz
    last_z = (100000 - 1) // wz

    def pick_z(a):
        return pl.BlockSpec(
            (8, wz), lambda i, a=a: (0, jnp.minimum((sz // wz) * a + i, last_z)))

    full = lambda shape: pl.BlockSpec(shape, lambda i: (0, 0))
    ztab2, = pl.pallas_call(
        _tp_zip_body,
        grid=(gz,),
        in_specs=[pick_z(a) for a in range(16)] + [full((128, 128))],
        out_specs=[pl.BlockSpec((wz, 128), lambda i: (i, 0))],
        out_shape=[jax.ShapeDtypeStruct((sz, 128), f32)],
        interpret=interpret,
    )(*([pz] * 16), eye)
    return ztab2


def _tc_body(g_u, g_i, g_z, hist, sidx,
             age_t, gen_t, occ_t, uctx_t, genre_t, year_t, ictx_t,
             w1u_big, w1z_big, w1u_age, w1u_gen, w1u_occ, w1u_ctx, w1u_hist,
             hist_W, hist_b, u_b1, u_W2, u_b2,
             w1i_big, w1i_genre, w1i_year, w1i_ictx, i_b1, i_W2, i_b2,
             out):
    f32 = jnp.float32
    bs = g_u.shape[0]
    idx = sidx[...]
    lane = lax.broadcasted_iota(jnp.int32, (bs, 128), 1)
    grp32 = lane >> 5
    grp8 = lane >> 3

    def emb(k, tab_ref, w_ref):
        n = tab_ref.shape[0]
        col = idx[:, k:k + 1]
        oh = (col == lax.broadcasted_iota(jnp.int32, (bs, n), 1)).astype(f32)
        fused = jnp.dot(tab_ref[...], w_ref[...], preferred_element_type=f32)
        return jnp.dot(oh, fused, preferred_element_type=f32)

    def pick(g_ref, grp, k):
        m = (grp == idx[:, k:k + 1]).astype(f32)
        return g_ref[...] * m

    h = jnp.dot(hist[...], hist_W[...], preferred_element_type=f32) + hist_b[...]
    au = jnp.dot(pick(g_u, grp32, 7), w1u_big[...], preferred_element_type=f32)
    au += jnp.dot(pick(g_z, grp8, 9), w1z_big[...], preferred_element_type=f32)
    au += emb(0, age_t, w1u_age)
    au += emb(1, gen_t, w1u_gen)
    au += emb(2, occ_t, w1u_occ)
    au += emb(3, uctx_t, w1u_ctx)
    au += jnp.dot(h, w1u_hist[...], preferred_element_type=f32)
    au += u_b1[...]
    au = jnp.maximum(au, 0.0)
    u = jnp.maximum(jnp.dot(au, u_W2[...], preferred_element_type=f32) + u_b2[...], 0.0)

    ai = jnp.dot(pick(g_i, grp32, 8), w1i_big[...], preferred_element_type=f32)
    ai += emb(4, genre_t, w1i_genre)
    ai += emb(5, year_t, w1i_year)
    ai += emb(6, ictx_t, w1i_ictx)
    ai += i_b1[...]
    ai = jnp.maximum(ai, 0.0)
    it = jnp.maximum(jnp.dot(ai, i_W2[...], preferred_element_type=f32) + i_b2[...], 0.0)

    ones = jnp.full((u.shape[1], 1), 1.0, f32)
    out[...] = jnp.dot(u * it, ones, preferred_element_type=f32)


def _tc_towers(g_u, g_i, g_z, history_emb, sidx, small_tabs, weights,
               bs=2048, interpret=False):
    grid = _B // bs
    f32 = jnp.float32

    def batched(d):
        return pl.BlockSpec((bs, d), lambda i: (i, 0))

    def full2(shape):
        return pl.BlockSpec(shape, lambda i: (0, 0))

    in_specs = (
        [batched(128), batched(128), batched(128), batched(history_emb.shape[1]),
         batched(16)]
        + [full2(t.shape) for t in small_tabs]
        + [full2(w.shape) for w in weights]
    )
    out_spec = pl.BlockSpec((bs, 1), lambda i: (i, 0))
    return pl.pallas_call(
        _tc_body,
        grid=(grid,),
        in_specs=in_specs,
        out_specs=out_spec,
        out_shape=jax.ShapeDtypeStruct((_B, 1), f32),
        interpret=interpret,
    )(g_u, g_i, g_z, history_emb, sidx, *small_tabs, *weights)


def kernel(user_id, age, gender, occupation, zipcode, context, item_id, genre,
           year, item_context, history_emb,
           user_id_emb, age_emb, gender_emb, occupation_emb, zip_emb,
           u_context_emb, hist_W, hist_b, u_W1, u_b1, u_W2, u_b2,
           item_id_emb, genre_emb, year_emb, i_context_emb,
           i_W1, i_b1, i_W2, i_b2):
    i32 = jnp.int32
    uid = user_id.astype(i32)
    iid = item_id.astype(i32)
    zid = zipcode.astype(i32)

    ztab2 = _transpose_zip(zip_emb)
    utab2, itab2 = _transpose_big(user_id_emb, item_id_emb)

    chunks = lambda a: a.reshape(32, 4, _CHUNK)
    idx_all = jnp.concatenate(
        [chunks(uid % _S_BIG), chunks(iid % _S_BIG), chunks(zid % _S_ZIP),
         jnp.zeros((32, 4, _CHUNK), i32)], axis=1)
    g_u, g_i, g_z = _sc_gather(idx_all, [utab2, itab2, ztab2])

    sidx = jnp.stack(
        [age.astype(i32), gender.astype(i32), occupation.astype(i32),
         context.astype(i32), genre.astype(i32), year.astype(i32),
         item_context.astype(i32), uid // _S_BIG, iid // _S_BIG, zid // _S_ZIP,
         jnp.zeros((_B,), i32), jnp.zeros((_B,), i32),
         jnp.zeros((_B,), i32), jnp.zeros((_B,), i32),
         jnp.zeros((_B,), i32), jnp.zeros((_B,), i32)], axis=1)

    small_tabs = [age_emb, gender_emb, occupation_emb, u_context_emb,
                  genre_emb, year_emb, i_context_emb]
    weights = [
        jnp.tile(u_W1[0:32], (4, 1)),
        jnp.tile(u_W1[52:60], (16, 1)),
        u_W1[32:40], u_W1[40:44], u_W1[44:52], u_W1[60:68], u_W1[68:84],
        hist_W, hist_b.reshape(1, -1), u_b1.reshape(1, -1), u_W2,
        u_b2.reshape(1, -1),
        jnp.tile(i_W1[0:32], (4, 1)),
        i_W1[32:40], i_W1[40:44], i_W1[44:52],
        i_b1.reshape(1, -1), i_W2, i_b2.reshape(1, -1),
    ]
    out = _tc_towers(g_u, g_i, g_z, history_emb, sidx, small_tabs, weights)
    return out.reshape(_B)


# speedup vs baseline: 1.1380x; 1.0142x over previous
import jax
import jax.numpy as jnp
from jax import lax
from jax.experimental import pallas as pl
from jax.experimental.pallas import tpu as pltpu
from jax.experimental.pallas import tpu_sc as plsc

_B = 16384
_CHUNK = 128


def _sc_gather(idx_all, tabs):
    info = plsc.get_sparse_core_info()
    nw = info.num_cores * info.num_subcores
    b_per_w = _B // nw
    n_chunks = b_per_w // _CHUNK
    n_tabs = len(tabs)
    idx_rows = idx_all.shape[1]

    mesh = plsc.VectorSubcoreMesh(core_axis_name="c", subcore_axis_name="s")

    def body(*refs):
        idx_hbm = refs[0]
        tab_hbms = refs[1:1 + n_tabs]
        outs = refs[1 + n_tabs:1 + 2 * n_tabs]
        idx_v, rows_v, sem = refs[1 + 2 * n_tabs:]
        wid = lax.axis_index("s") * info.num_cores + lax.axis_index("c")
        base = wid * b_per_w
        pltpu.sync_copy(idx_hbm.at[wid], idx_v)
        out = pl.ds(base, b_per_w)
        for t in range(n_tabs):
            copies = []
            for j in range(n_chunks):
                copies.append(pltpu.async_copy(
                    tab_hbms[t].at[idx_v.at[n_chunks * t + j]],
                    rows_v.at[pl.ds(j * _CHUNK, _CHUNK)], sem))
            for c in copies:
                c.wait()
            pltpu.sync_copy(rows_v, outs[t].at[out])

    f32 = jnp.float32
    run = pl.kernel(
        body,
        out_type=tuple(jax.ShapeDtypeStruct((_B, 128), f32)
                       for _ in range(n_tabs)),
        mesh=mesh,
        scratch_types=[
            pltpu.VMEM((idx_rows, _CHUNK), jnp.int32),
            pltpu.VMEM((b_per_w, 128), f32),
            pltpu.SemaphoreType.DMA,
        ],
    )
    return run(idx_all, *tabs)


_S_BIG = 253952
_S_ZIP = 6656
_W_BIG = 8192
_W_ZIP = 512


def _stack_transpose(p_refs, eye_ref, s, w, n):
    k = len(p_refs)
    pieces = [r[...] for r in p_refs]
    bound = n - (k - 1) * s - w * pl.program_id(0)
    cols = jax.lax.broadcasted_iota(jnp.int32, pieces[-1].shape, 1)
    pieces[-1] = jnp.where(cols < bound, pieces[-1], 0.0)
    x = jnp.concatenate(pieces, axis=0)
    return jax.lax.dot_general(x, eye_ref[...], (((0,), (0,)), ((), ())),
                               preferred_element_type=jnp.float32)


def _tp_big_body(*args):
    pus, pis = args[0:4], args[4:8]
    eye_ref, ou_ref, oi_ref = args[8], args[9], args[10]
    ou_ref[...] = _stack_transpose(pus, eye_ref, _S_BIG, _W_BIG, 1000000)
    oi_ref[...] = _stack_transpose(pis, eye_ref, _S_BIG, _W_BIG, 1000000)


def _tp_zip_body(*args):
    pzs, eye_ref, oz_ref = args[0:16], args[16], args[17]
    oz_ref[...] = _stack_transpose(pzs, eye_ref, _S_ZIP, _W_ZIP, 100000)


def _transpose_big(utab, itab, interpret=False):
    f32 = jnp.float32
    pu = utab.T
    pi = itab.T
    eye = jnp.eye(128, dtype=f32)
    s, w = _S_BIG, _W_BIG
    grid = s // w
    last_blk = (1000000 - 1) // w

    def pick_block(a):
        return pl.BlockSpec(
            (32, w), lambda i, a=a: (0, jnp.minimum((s // w) * a + i, last_blk)))

    full = lambda shape: pl.BlockSpec(shape, lambda i: (0, 0))
    out = pl.pallas_call(
        _tp_big_body,
        grid=(grid,),
        in_specs=[pick_block(a) for a in range(4)] * 2 + [full((128, 128))],
        out_specs=[pl.BlockSpec((w, 128), lambda i: (i, 0)),
                   pl.BlockSpec((w, 128), lambda i: (i, 0))],
        out_shape=[jax.ShapeDtypeStruct((s, 128), f32),
                   jax.ShapeDtypeStruct((s, 128), f32)],
        interpret=interpret,
    )(pu, pu, pu, pu, pi, pi, pi, pi, eye)
    return out


def _transpose_zip(zip_emb, interpret=False):
    f32 = jnp.float32
    pz = zip_emb.T
    eye = jnp.eye(128, dtype=f32)
    sz, wz = _S_ZIP, _W_ZIP
    gz = sz // wz
    last_z = (100000 - 1) // wz

    def pick_z(a):
        return pl.BlockSpec(
            (8, wz), lambda i, a=a: (0, jnp.minimum((sz // wz) * a + i, last_z)))

    full = lambda shape: pl.BlockSpec(shape, lambda i: (0, 0))
    ztab2, = pl.pallas_call(
        _tp_zip_body,
        grid=(gz,),
        in_specs=[pick_z(a) for a in range(16)] + [full((128, 128))],
        out_specs=[pl.BlockSpec((wz, 128), lambda i: (i, 0))],
        out_shape=[jax.ShapeDtypeStruct((sz, 128), f32)],
        interpret=interpret,
    )(*([pz] * 16), eye)
    return ztab2


def _tc_body(g_u, g_i, g_z, hist, sidx,
             age_t, gen_t, occ_t, uctx_t, genre_t, year_t, ictx_t,
             w1u_big, w1z_big, w1u_age, w1u_gen, w1u_occ, w1u_ctx, w1u_hist,
             hist_W, hist_b, u_b1, u_W2, u_b2,
             w1i_big, w1i_genre, w1i_year, w1i_ictx, i_b1, i_W2, i_b2,
             out):
    f32 = jnp.float32
    bs = g_u.shape[0]
    idx = sidx[...]
    lane = lax.broadcasted_iota(jnp.int32, (bs, 128), 1)
    grp32 = lane >> 5
    grp8 = lane >> 3

    def emb(k, tab_ref, w_ref):
        n = tab_ref.shape[0]
        col = idx[:, k:k + 1]
        oh = (col == lax.broadcasted_iota(jnp.int32, (bs, n), 1)).astype(f32)
        fused = jnp.dot(tab_ref[...], w_ref[...], preferred_element_type=f32)
        return jnp.dot(oh, fused, preferred_element_type=f32)

    def pick(g_ref, grp, k):
        m = (grp == idx[:, k:k + 1]).astype(f32)
        return g_ref[...] * m

    h = jnp.dot(hist[...], hist_W[...], preferred_element_type=f32) + hist_b[...]
    au = jnp.dot(pick(g_u, grp32, 7), w1u_big[...], preferred_element_type=f32)
    au += jnp.dot(pick(g_z, grp8, 9), w1z_big[...], preferred_element_type=f32)
    au += emb(0, age_t, w1u_age)
    au += emb(1, gen_t, w1u_gen)
    au += emb(2, occ_t, w1u_occ)
    au += emb(3, uctx_t, w1u_ctx)
    au += jnp.dot(h, w1u_hist[...], preferred_element_type=f32)
    au += u_b1[...]
    au = jnp.maximum(au, 0.0)
    u = jnp.maximum(jnp.dot(au, u_W2[...], preferred_element_type=f32) + u_b2[...], 0.0)

    ai = jnp.dot(pick(g_i, grp32, 8), w1i_big[...], preferred_element_type=f32)
    ai += emb(4, genre_t, w1i_genre)
    ai += emb(5, year_t, w1i_year)
    ai += emb(6, ictx_t, w1i_ictx)
    ai += i_b1[...]
    ai = jnp.maximum(ai, 0.0)
    it = jnp.maximum(jnp.dot(ai, i_W2[...], preferred_element_type=f32) + i_b2[...], 0.0)

    ones = jnp.full((u.shape[1], 1), 1.0, f32)
    out[...] = jnp.dot(u * it, ones, preferred_element_type=f32)


def _tc_towers(g_u, g_i, g_z, history_emb, sidx, small_tabs, weights,
               bs=2048, interpret=False):
    grid = _B // bs
    f32 = jnp.float32

    def batched(d):
        return pl.BlockSpec((bs, d), lambda i: (i, 0))

    def full2(shape):
        return pl.BlockSpec(shape, lambda i: (0, 0))

    in_specs = (
        [batched(128), batched(128), batched(128), batched(history_emb.shape[1]),
         batched(16)]
        + [full2(t.shape) for t in small_tabs]
        + [full2(w.shape) for w in weights]
    )
    out_spec = pl.BlockSpec((bs, 1), lambda i: (i, 0))
    return pl.pallas_call(
        _tc_body,
        grid=(grid,),
        in_specs=in_specs,
        out_specs=out_spec,
        out_shape=jax.ShapeDtypeStruct((_B, 1), f32),
        interpret=interpret,
    )(g_u, g_i, g_z, history_emb, sidx, *small_tabs, *weights)


def kernel(user_id, age, gender, occupation, zipcode, context, item_id, genre,
           year, item_context, history_emb,
           user_id_emb, age_emb, gender_emb, occupation_emb, zip_emb,
           u_context_emb, hist_W, hist_b, u_W1, u_b1, u_W2, u_b2,
           item_id_emb, genre_emb, year_emb, i_context_emb,
           i_W1, i_b1, i_W2, i_b2):
    i32 = jnp.int32
    uid = user_id.astype(i32)
    iid = item_id.astype(i32)
    zid = zipcode.astype(i32)

    ztab2 = _transpose_zip(zip_emb)
    utab2, itab2 = _transpose_big(user_id_emb, item_id_emb)

    chunks = lambda a: a.reshape(32, 4, _CHUNK)
    idx_all = jnp.concatenate(
        [chunks(uid % _S_BIG), chunks(iid % _S_BIG), chunks(zid % _S_ZIP),
         jnp.zeros((32, 4, _CHUNK), i32)], axis=1)
    g_u, g_i, g_z = _sc_gather(idx_all, [utab2, itab2, ztab2])

    sidx = jnp.stack(
        [age.astype(i32), gender.astype(i32), occupation.astype(i32),
         context.astype(i32), genre.astype(i32), year.astype(i32),
         item_context.astype(i32), uid // _S_BIG, iid // _S_BIG, zid // _S_ZIP,
         jnp.zeros((_B,), i32), jnp.zeros((_B,), i32),
         jnp.zeros((_B,), i32), jnp.zeros((_B,), i32),
         jnp.zeros((_B,), i32), jnp.zeros((_B,), i32)], axis=1)

    small_tabs = [age_emb, gender_emb, occupation_emb, u_context_emb,
                  genre_emb, year_emb, i_context_emb]
    weights = [
        jnp.tile(u_W1[0:32], (4, 1)),
        jnp.tile(u_W1[52:60], (16, 1)),
        u_W1[32:40], u_W1[40:44], u_W1[44:52], u_W1[60:68], u_W1[68:84],
        hist_W, hist_b.reshape(1, -1), u_b1.reshape(1, -1), u_W2,
        u_b2.reshape(1, -1),
        jnp.tile(i_W1[0:32], (4, 1)),
        i_W1[32:40], i_W1[40:44], i_W1[44:52],
        i_b1.reshape(1, -1), i_W2, i_b2.reshape(1, -1),
    ]
    out = _tc_towers(g_u, g_i, g_z, history_emb, sidx, small_tabs, weights)
    return out.reshape(_B)
